# baseline (speedup 1.0000x reference)
import jax
import jax.numpy as jnp
from jax import lax
from jax.experimental import pallas as pl
from jax.experimental.pallas import tpu as pltpu
from jax.experimental.pallas import tpu_sc as plsc

_N = 8192
_E = 262144
_NC = 2
_NS = 16
_L = 16
_NW = _NC * _NS
_CHUNK = _E // _NW


def _gather_body(edges_hbm, score_hbm, out_hbm, edges_v, out_v, sem):
    wid = lax.axis_index("s") * _NC + lax.axis_index("c")
    base = wid * _CHUNK
    pltpu.sync_copy(edges_hbm.at[pl.ds(base * 2, _CHUNK * 2)], edges_v)

    lane = lax.iota(jnp.int32, _L)

    def step(k, carry):
        b = k * _L
        pos = (b + lane) * 2
        r = plsc.load_gather(edges_v, [pos])
        c = plsc.load_gather(edges_v, [pos + 1])
        idx = ((r >> 3) << 16) | ((c >> 7) << 10) | ((r & 7) << 7) | (c & 127)
        pltpu.async_copy(score_hbm.at[idx], out_v.at[pl.ds(b, _L)], sem)
        return carry

    lax.fori_loop(0, _CHUNK // _L, step, 0)
    pltpu.make_async_copy(score_hbm.at[pl.ds(0, _CHUNK)], out_v, sem).wait()
    pltpu.sync_copy(out_v, out_hbm.at[pl.ds(base, _CHUNK)])


def kernel(inputs, edges, score_all):
    del inputs
    edges_flat = edges.astype(jnp.int32).reshape(-1)
    score_phys = (
        score_all.reshape(1024, 8, 64, 128)
        .transpose(0, 2, 1, 3)
        .reshape(_N * _N)
    )
    mesh = plsc.VectorSubcoreMesh(
        core_axis_name="c", subcore_axis_name="s",
        num_cores=_NC, num_subcores=_NS,
    )
    run = pl.kernel(
        _gather_body,
        out_type=jax.ShapeDtypeStruct((_E,), jnp.float32),
        mesh=mesh,
        compiler_params=pltpu.CompilerParams(needs_layout_passes=False),
        scratch_types=[
            pltpu.VMEM((_CHUNK * 2,), jnp.int32),
            pltpu.VMEM((_CHUNK,), jnp.float32),
            pltpu.SemaphoreType.DMA,
        ],
    )
    return run(edges_flat, score_phys)
